# initial kernel scaffold (unmeasured)
import jax
import jax.numpy as jnp
from jax import lax
from jax.experimental import pallas as pl
from jax.experimental.pallas import tpu as pltpu

N_DEV = 4


def kernel(x, w_mat):
    m_total, k_local = x.shape
    _, n = w_mat.shape
    m_per = m_total // N_DEV

    def body(x_ref, w_ref, out_ref, acc_ref, comm_ref, send_sems, recv_sems):
        q = lax.axis_index("i")
        left = lax.rem(q + N_DEV - 1, N_DEV)
        right = lax.rem(q + 1, N_DEV)

        barrier_sem = pltpu.get_barrier_semaphore()
        for nbr in (left, right):
            pl.semaphore_signal(
                barrier_sem, inc=1,
                device_id=(nbr,), device_id_type=pl.DeviceIdType.MESH,
            )
        pl.semaphore_wait(barrier_sem, 2)

        def chunk_partial(c):
            return jnp.dot(
                x_ref[pl.ds(c * m_per, m_per), :], w_ref[:, :],
                preferred_element_type=jnp.float32,
            )

        acc_ref[:, :] = chunk_partial(left)

        for s in range(N_DEV - 1):
            rdma = pltpu.make_async_remote_copy(
                src_ref=acc_ref,
                dst_ref=comm_ref.at[s],
                send_sem=send_sems.at[s],
                recv_sem=recv_sems.at[s],
                device_id=(right,),
                device_id_type=pl.DeviceIdType.MESH,
            )
            rdma.start()
            c_next = lax.rem(q + 2 * N_DEV - s - 2, N_DEV)
            partial = chunk_partial(c_next)
            rdma.wait()
            if s < N_DEV - 2:
                acc_ref[:, :] = comm_ref[s] + partial
            else:
                y = comm_ref[s] + partial
                out_ref[:, :] = y * jax.nn.sigmoid(y)

    return pl.pallas_call(
        body,
        out_shape=jax.ShapeDtypeStruct((m_per, n), jnp.float32),
        in_specs=[
            pl.BlockSpec(memory_space=pltpu.VMEM),
            pl.BlockSpec(memory_space=pltpu.VMEM),
        ],
        out_specs=pl.BlockSpec(memory_space=pltpu.VMEM),
        scratch_shapes=[
            pltpu.VMEM((m_per, n), jnp.float32),
            pltpu.VMEM((N_DEV - 1, m_per, n), jnp.float32),
            pltpu.SemaphoreType.DMA((N_DEV - 1,)),
            pltpu.SemaphoreType.DMA((N_DEV - 1,)),
        ],
        compiler_params=pltpu.CompilerParams(collective_id=0),
    )(x, w_mat)


# baseline (device time: 304807 ns/iter reference)
import jax
import jax.numpy as jnp
from jax import lax
from jax.experimental import pallas as pl
from jax.experimental.pallas import tpu as pltpu

N_DEV = 4


def kernel(x, w_mat):
    m_total, k_local = x.shape
    _, n = w_mat.shape
    m_per = m_total // N_DEV

    def body(x_ref, w_ref, out_ref, comm_ref, send_sems, recv_sems, credit_sem):
        q = lax.axis_index("i")
        left = lax.rem(q + N_DEV - 1, N_DEV)
        right = lax.rem(q + 1, N_DEV)

        barrier_sem = pltpu.get_barrier_semaphore()
        for nbr in (left, right):
            pl.semaphore_signal(
                barrier_sem, inc=1,
                device_id=(nbr,), device_id_type=pl.DeviceIdType.MESH,
            )
        pl.semaphore_wait(barrier_sem, 2)

        def chunk_partial(c):
            return jnp.dot(
                x_ref[pl.ds(c * m_per, m_per), :], w_ref[:, :],
                preferred_element_type=jnp.float32,
            )

        out_ref[:, :] = chunk_partial(left)

        for s in range(N_DEV - 1):
            slot = s % 2
            if s == 2:
                pl.semaphore_wait(credit_sem, 1)
            rdma = pltpu.make_async_remote_copy(
                src_ref=out_ref,
                dst_ref=comm_ref.at[slot],
                send_sem=send_sems.at[s],
                recv_sem=recv_sems.at[s],
                device_id=(right,),
                device_id_type=pl.DeviceIdType.MESH,
            )
            rdma.start()
            c_next = lax.rem(q + 2 * N_DEV - s - 2, N_DEV)
            partial = chunk_partial(c_next)
            rdma.wait()
            if s < N_DEV - 2:
                out_ref[:, :] = comm_ref[slot] + partial
                if s == 0:
                    pl.semaphore_signal(
                        credit_sem, inc=1,
                        device_id=(left,), device_id_type=pl.DeviceIdType.MESH,
                    )
            else:
                y = comm_ref[slot] + partial
                out_ref[:, :] = y * jax.nn.sigmoid(y)

    return pl.pallas_call(
        body,
        out_shape=jax.ShapeDtypeStruct((m_per, n), jnp.float32),
        in_specs=[
            pl.BlockSpec(memory_space=pltpu.VMEM),
            pl.BlockSpec(memory_space=pltpu.VMEM),
        ],
        out_specs=pl.BlockSpec(memory_space=pltpu.VMEM),
        scratch_shapes=[
            pltpu.VMEM((2, m_per, n), jnp.float32),
            pltpu.SemaphoreType.DMA((N_DEV - 1,)),
            pltpu.SemaphoreType.DMA((N_DEV - 1,)),
            pltpu.SemaphoreType.REGULAR,
        ],
        compiler_params=pltpu.CompilerParams(
            collective_id=0,
            vmem_limit_bytes=120 * 1024 * 1024,
        ),
    )(x, w_mat)


# device time: 169943 ns/iter; 1.7936x vs baseline; 1.7936x over previous
import jax
import jax.numpy as jnp
from jax import lax
from jax.experimental import pallas as pl
from jax.experimental.pallas import tpu as pltpu

N_DEV = 4


def kernel(x, w_mat):
    m_total, k_local = x.shape
    _, n = w_mat.shape
    m_per = m_total // N_DEV
    n2 = n // 2

    def body(x_ref, w_ref, out_ref, comm_a, comm_b,
             send_a, recv_a, send_b, recv_b, credit_a, credit_b):
        q = lax.axis_index("i")
        left = lax.rem(q + N_DEV - 1, N_DEV)
        right = lax.rem(q + 1, N_DEV)

        barrier_sem = pltpu.get_barrier_semaphore()
        for nbr in (left, right):
            pl.semaphore_signal(
                barrier_sem, inc=1,
                device_id=(nbr,), device_id_type=pl.DeviceIdType.MESH,
            )
        pl.semaphore_wait(barrier_sem, 2)

        def partial(c, lo, sz):
            return jnp.dot(
                x_ref[pl.ds(c * m_per, m_per), :], w_ref[:, lo:lo + sz],
                preferred_element_type=jnp.float32,
            )

        out_ref[:, 0:n2] = partial(left, 0, n2)
        out_ref[:, n2:n] = partial(right, n2, n2)

        for s in range(N_DEV - 1):
            slot = s % 2
            if s == 2:
                pl.semaphore_wait(credit_a, 1)
                pl.semaphore_wait(credit_b, 1)
            rdma_a = pltpu.make_async_remote_copy(
                src_ref=out_ref.at[:, pl.ds(0, n2)],
                dst_ref=comm_a.at[slot],
                send_sem=send_a.at[s],
                recv_sem=recv_a.at[s],
                device_id=(right,),
                device_id_type=pl.DeviceIdType.MESH,
            )
            rdma_b = pltpu.make_async_remote_copy(
                src_ref=out_ref.at[:, pl.ds(n2, n2)],
                dst_ref=comm_b.at[slot],
                send_sem=send_b.at[s],
                recv_sem=recv_b.at[s],
                device_id=(left,),
                device_id_type=pl.DeviceIdType.MESH,
            )
            rdma_a.start()
            rdma_b.start()

            c_a = lax.rem(q + 2 * N_DEV - s - 2, N_DEV)
            c_b = lax.rem(q + s + 2, N_DEV)
            if s % 2 == 0:
                full = partial(c_a, 0, n)
                p_a = full[:, 0:n2]
                p_b = full[:, n2:n]
            else:
                p_a = partial(c_a, 0, n2)
                p_b = partial(c_b, n2, n2)

            rdma_a.wait()
            if s < N_DEV - 2:
                out_ref[:, 0:n2] = comm_a[slot] + p_a
            else:
                y_a = comm_a[slot] + p_a
                out_ref[:, 0:n2] = y_a * jax.nn.sigmoid(y_a)
            if s == 0:
                pl.semaphore_signal(
                    credit_a, inc=1,
                    device_id=(left,), device_id_type=pl.DeviceIdType.MESH,
                )

            rdma_b.wait()
            if s < N_DEV - 2:
                out_ref[:, n2:n] = comm_b[slot] + p_b
            else:
                y_b = comm_b[slot] + p_b
                out_ref[:, n2:n] = y_b * jax.nn.sigmoid(y_b)
            if s == 0:
                pl.semaphore_signal(
                    credit_b, inc=1,
                    device_id=(right,), device_id_type=pl.DeviceIdType.MESH,
                )

    return pl.pallas_call(
        body,
        out_shape=jax.ShapeDtypeStruct((m_per, n), jnp.float32),
        in_specs=[
            pl.BlockSpec(memory_space=pltpu.VMEM),
            pl.BlockSpec(memory_space=pltpu.VMEM),
        ],
        out_specs=pl.BlockSpec(memory_space=pltpu.VMEM),
        scratch_shapes=[
            pltpu.VMEM((2, m_per, n2), jnp.float32),
            pltpu.VMEM((2, m_per, n2), jnp.float32),
            pltpu.SemaphoreType.DMA((N_DEV - 1,)),
            pltpu.SemaphoreType.DMA((N_DEV - 1,)),
            pltpu.SemaphoreType.DMA((N_DEV - 1,)),
            pltpu.SemaphoreType.DMA((N_DEV - 1,)),
            pltpu.SemaphoreType.REGULAR,
            pltpu.SemaphoreType.REGULAR,
        ],
        compiler_params=pltpu.CompilerParams(
            collective_id=0,
            vmem_limit_bytes=120 * 1024 * 1024,
        ),
    )(x, w_mat)


# device time: 161430 ns/iter; 1.8882x vs baseline; 1.0527x over previous
import jax
import jax.numpy as jnp
from jax import lax
from jax.experimental import pallas as pl
from jax.experimental.pallas import tpu as pltpu

N_DEV = 4
N_HOP = N_DEV - 1
SUB = 2


def kernel(x, w_mat):
    m_total, k_local = x.shape
    _, n = w_mat.shape
    m_per = m_total // N_DEV
    n2 = n // 2
    m_sub = m_per // SUB

    def body(x_ref, w_ref, out_ref, comm_a, comm_b,
             send_a, recv_a, send_b, recv_b, credit_a, credit_b):
        q = lax.axis_index("i")
        left = lax.rem(q + N_DEV - 1, N_DEV)
        right = lax.rem(q + 1, N_DEV)

        barrier_sem = pltpu.get_barrier_semaphore()
        for nbr in (left, right):
            pl.semaphore_signal(
                barrier_sem, inc=1,
                device_id=(nbr,), device_id_type=pl.DeviceIdType.MESH,
            )
        pl.semaphore_wait(barrier_sem, 2)

        def part(c, sub, lo, sz):
            return jnp.dot(
                x_ref[pl.ds(c * m_per + sub * m_sub, m_sub), :],
                w_ref[:, lo:lo + sz],
                preferred_element_type=jnp.float32,
            )

        def make_rdma(ring, s, sub):
            slot = s % 2
            r0 = sub * m_sub
            if ring == "a":
                return pltpu.make_async_remote_copy(
                    src_ref=out_ref.at[pl.ds(r0, m_sub), pl.ds(0, n2)],
                    dst_ref=comm_a.at[slot, pl.ds(r0, m_sub), :],
                    send_sem=send_a.at[s, sub],
                    recv_sem=recv_a.at[s, sub],
                    device_id=(right,),
                    device_id_type=pl.DeviceIdType.MESH,
                )
            return pltpu.make_async_remote_copy(
                src_ref=out_ref.at[pl.ds(r0, m_sub), pl.ds(n2, n2)],
                dst_ref=comm_b.at[slot, pl.ds(r0, m_sub), :],
                send_sem=send_b.at[s, sub],
                recv_sem=recv_b.at[s, sub],
                device_id=(left,),
                device_id_type=pl.DeviceIdType.MESH,
            )

        rdmas = {}

        def start(ring, s, sub):
            r = make_rdma(ring, s, sub)
            r.start()
            rdmas[(ring, s, sub)] = r

        out_ref[0:m_sub, 0:n2] = part(left, 0, 0, n2)
        start("a", 0, 0)
        out_ref[0:m_sub, n2:n] = part(right, 0, n2, n2)
        start("b", 0, 0)
        out_ref[m_sub:m_per, 0:n2] = part(left, 1, 0, n2)
        start("a", 0, 1)
        out_ref[m_sub:m_per, n2:n] = part(right, 1, n2, n2)
        start("b", 0, 1)

        for s in range(N_HOP):
            slot = s % 2
            last = s == N_HOP - 1
            c_a = lax.rem(q + 2 * N_DEV - s - 2, N_DEV)
            c_b = lax.rem(q + s + 2, N_DEV)
            if s % 2 == 0:
                f0 = part(c_a, 0, 0, n)
                pa0, pb0 = f0[:, 0:n2], f0[:, n2:n]
                f1 = part(c_a, 1, 0, n)
                pa1, pb1 = f1[:, 0:n2], f1[:, n2:n]
            else:
                pa0 = part(c_a, 0, 0, n2)
                pb0 = part(c_b, 0, n2, n2)
                pa1 = part(c_a, 1, 0, n2)
                pb1 = part(c_b, 1, n2, n2)

            def step(ring, sub, comm, p, col0, credit, credit_peer):
                r0 = sub * m_sub
                rdmas[(ring, s, sub)].wait()
                v = comm[slot, pl.ds(r0, m_sub), :] + p
                if last:
                    v = v * jax.nn.sigmoid(v)
                out_ref[r0:r0 + m_sub, col0:col0 + n2] = v
                if s == 0 and sub == SUB - 1:
                    pl.semaphore_signal(
                        credit, inc=1,
                        device_id=(credit_peer,),
                        device_id_type=pl.DeviceIdType.MESH,
                    )
                if not last:
                    if s == 1 and sub == 0:
                        pl.semaphore_wait(credit, 1)
                    start(ring, s + 1, sub)

            step("a", 0, comm_a, pa0, 0, credit_a, left)
            step("b", 0, comm_b, pb0, n2, credit_b, right)
            step("a", 1, comm_a, pa1, 0, credit_a, left)
            step("b", 1, comm_b, pb1, n2, credit_b, right)

    return pl.pallas_call(
        body,
        out_shape=jax.ShapeDtypeStruct((m_per, n), jnp.float32),
        in_specs=[
            pl.BlockSpec(memory_space=pltpu.VMEM),
            pl.BlockSpec(memory_space=pltpu.VMEM),
        ],
        out_specs=pl.BlockSpec(memory_space=pltpu.VMEM),
        scratch_shapes=[
            pltpu.VMEM((2, m_per, n2), jnp.float32),
            pltpu.VMEM((2, m_per, n2), jnp.float32),
            pltpu.SemaphoreType.DMA((N_HOP, SUB)),
            pltpu.SemaphoreType.DMA((N_HOP, SUB)),
            pltpu.SemaphoreType.DMA((N_HOP, SUB)),
            pltpu.SemaphoreType.DMA((N_HOP, SUB)),
            pltpu.SemaphoreType.REGULAR,
            pltpu.SemaphoreType.REGULAR,
        ],
        compiler_params=pltpu.CompilerParams(
            collective_id=0,
            vmem_limit_bytes=120 * 1024 * 1024,
        ),
    )(x, w_mat)


# device time: 160153 ns/iter; 1.9032x vs baseline; 1.0080x over previous
import jax
import jax.numpy as jnp
from jax import lax
from jax.experimental import pallas as pl
from jax.experimental.pallas import tpu as pltpu

N_DEV = 4
N_HOP = N_DEV - 1
SUB = 4


def kernel(x, w_mat):
    m_total, k_local = x.shape
    _, n = w_mat.shape
    m_per = m_total // N_DEV
    n2 = n // 2
    m_sub = m_per // SUB

    def body(x_ref, w_ref, out_ref, comm_a, comm_b,
             send_a, recv_a, send_b, recv_b, credit_a, credit_b):
        q = lax.axis_index("i")
        left = lax.rem(q + N_DEV - 1, N_DEV)
        right = lax.rem(q + 1, N_DEV)

        barrier_sem = pltpu.get_barrier_semaphore()
        for nbr in (left, right):
            pl.semaphore_signal(
                barrier_sem, inc=1,
                device_id=(nbr,), device_id_type=pl.DeviceIdType.MESH,
            )
        pl.semaphore_wait(barrier_sem, 2)

        def part(c, sub, lo, sz):
            return jnp.dot(
                x_ref[pl.ds(c * m_per + sub * m_sub, m_sub), :],
                w_ref[:, lo:lo + sz],
                preferred_element_type=jnp.float32,
            )

        def make_rdma(ring, s, sub):
            slot = s % 2
            r0 = sub * m_sub
            if ring == "a":
                return pltpu.make_async_remote_copy(
                    src_ref=out_ref.at[pl.ds(r0, m_sub), pl.ds(0, n2)],
                    dst_ref=comm_a.at[slot, pl.ds(r0, m_sub), :],
                    send_sem=send_a.at[s, sub],
                    recv_sem=recv_a.at[s, sub],
                    device_id=(right,),
                    device_id_type=pl.DeviceIdType.MESH,
                )
            return pltpu.make_async_remote_copy(
                src_ref=out_ref.at[pl.ds(r0, m_sub), pl.ds(n2, n2)],
                dst_ref=comm_b.at[slot, pl.ds(r0, m_sub), :],
                send_sem=send_b.at[s, sub],
                recv_sem=recv_b.at[s, sub],
                device_id=(left,),
                device_id_type=pl.DeviceIdType.MESH,
            )

        rdmas = {}

        def start(ring, s, sub):
            r = make_rdma(ring, s, sub)
            r.start()
            rdmas[(ring, s, sub)] = r

        for sub in range(SUB):
            r0 = sub * m_sub
            out_ref[r0:r0 + m_sub, 0:n2] = part(left, sub, 0, n2)
            start("a", 0, sub)
            out_ref[r0:r0 + m_sub, n2:n] = part(right, sub, n2, n2)
            start("b", 0, sub)

        for s in range(N_HOP):
            slot = s % 2
            last = s == N_HOP - 1
            c_a = lax.rem(q + 2 * N_DEV - s - 2, N_DEV)
            c_b = lax.rem(q + s + 2, N_DEV)
            pa, pb = [], []
            for sub in range(SUB):
                if s % 2 == 0:
                    f = part(c_a, sub, 0, n)
                    pa.append(f[:, 0:n2])
                    pb.append(f[:, n2:n])
                else:
                    pa.append(part(c_a, sub, 0, n2))
                    pb.append(part(c_b, sub, n2, n2))

            def step(ring, sub, comm, p, col0, credit, credit_peer):
                r0 = sub * m_sub
                rdmas[(ring, s, sub)].wait()
                v = comm[slot, pl.ds(r0, m_sub), :] + p
                if last:
                    v = v * jax.nn.sigmoid(v)
                out_ref[r0:r0 + m_sub, col0:col0 + n2] = v
                if s == 0 and sub == SUB - 1:
                    pl.semaphore_signal(
                        credit, inc=1,
                        device_id=(credit_peer,),
                        device_id_type=pl.DeviceIdType.MESH,
                    )
                if not last:
                    if s == 1 and sub == 0:
                        pl.semaphore_wait(credit, 1)
                    start(ring, s + 1, sub)

            for sub in range(SUB):
                step("a", sub, comm_a, pa[sub], 0, credit_a, left)
                step("b", sub, comm_b, pb[sub], n2, credit_b, right)

    return pl.pallas_call(
        body,
        out_shape=jax.ShapeDtypeStruct((m_per, n), jnp.float32),
        in_specs=[
            pl.BlockSpec(memory_space=pltpu.VMEM),
            pl.BlockSpec(memory_space=pltpu.VMEM),
        ],
        out_specs=pl.BlockSpec(memory_space=pltpu.VMEM),
        scratch_shapes=[
            pltpu.VMEM((2, m_per, n2), jnp.float32),
            pltpu.VMEM((2, m_per, n2), jnp.float32),
            pltpu.SemaphoreType.DMA((N_HOP, SUB)),
            pltpu.SemaphoreType.DMA((N_HOP, SUB)),
            pltpu.SemaphoreType.DMA((N_HOP, SUB)),
            pltpu.SemaphoreType.DMA((N_HOP, SUB)),
            pltpu.SemaphoreType.REGULAR,
            pltpu.SemaphoreType.REGULAR,
        ],
        compiler_params=pltpu.CompilerParams(
            collective_id=0,
            vmem_limit_bytes=120 * 1024 * 1024,
        ),
    )(x, w_mat)


# device time: 92526 ns/iter; 3.2943x vs baseline; 1.7309x over previous
import jax
import jax.numpy as jnp
from jax import lax
from jax.experimental import pallas as pl
from jax.experimental.pallas import tpu as pltpu

N_DEV = 4
N_HOP = N_DEV - 1
SUB = 4


def kernel(x, w_mat):
    m_total, k_local = x.shape
    _, n = w_mat.shape
    m_per = m_total // N_DEV
    n2 = n // 2
    m_sub = m_per // SUB

    def body(x_ref, w_ref, out_ref, comm_a, comm_b, stage_a, stage_b,
             send_a, recv_a, send_b, recv_b, credit_a, credit_b):
        q = lax.axis_index("i")
        left = lax.rem(q + N_DEV - 1, N_DEV)
        right = lax.rem(q + 1, N_DEV)

        barrier_sem = pltpu.get_barrier_semaphore()
        for nbr in (left, right):
            pl.semaphore_signal(
                barrier_sem, inc=1,
                device_id=(nbr,), device_id_type=pl.DeviceIdType.MESH,
            )
        pl.semaphore_wait(barrier_sem, 2)

        def part(c, sub, lo, sz):
            return jnp.dot(
                x_ref[pl.ds(c * m_per + sub * m_sub, m_sub), :],
                w_ref[:, lo:lo + sz],
                preferred_element_type=jnp.float32,
            )

        def make_rdma(ring, s, sub):
            slot = s % 2
            r0 = sub * m_sub
            if ring == "a":
                return pltpu.make_async_remote_copy(
                    src_ref=stage_a.at[pl.ds(r0, m_sub), :],
                    dst_ref=comm_a.at[slot, pl.ds(r0, m_sub), :],
                    send_sem=send_a.at[s, sub],
                    recv_sem=recv_a.at[s, sub],
                    device_id=(right,),
                    device_id_type=pl.DeviceIdType.MESH,
                )
            return pltpu.make_async_remote_copy(
                src_ref=stage_b.at[pl.ds(r0, m_sub), :],
                dst_ref=comm_b.at[slot, pl.ds(r0, m_sub), :],
                send_sem=send_b.at[s, sub],
                recv_sem=recv_b.at[s, sub],
                device_id=(left,),
                device_id_type=pl.DeviceIdType.MESH,
            )

        rdmas = {}

        def start(ring, s, sub):
            r = make_rdma(ring, s, sub)
            r.start()
            rdmas[(ring, s, sub)] = r

        for sub in range(SUB):
            r0 = sub * m_sub
            stage_a[r0:r0 + m_sub, :] = part(left, sub, 0, n2).astype(
                jnp.bfloat16)
            start("a", 0, sub)
            stage_b[r0:r0 + m_sub, :] = part(right, sub, n2, n2).astype(
                jnp.bfloat16)
            start("b", 0, sub)

        for s in range(N_HOP):
            slot = s % 2
            last = s == N_HOP - 1
            c_a = lax.rem(q + 2 * N_DEV - s - 2, N_DEV)
            c_b = lax.rem(q + s + 2, N_DEV)
            pa, pb = [], []
            for sub in range(SUB):
                if s % 2 == 0:
                    f = part(c_a, sub, 0, n)
                    pa.append(f[:, 0:n2])
                    pb.append(f[:, n2:n])
                else:
                    pa.append(part(c_a, sub, 0, n2))
                    pb.append(part(c_b, sub, n2, n2))

            def step(ring, sub, comm, stage, p, col0, credit, credit_peer):
                r0 = sub * m_sub
                rdmas[(ring, s, sub)].wait()
                v = comm[slot, pl.ds(r0, m_sub), :].astype(jnp.float32) + p
                if last:
                    v = v * jax.nn.sigmoid(v)
                    out_ref[r0:r0 + m_sub, col0:col0 + n2] = v
                else:
                    stage[r0:r0 + m_sub, :] = v.astype(jnp.bfloat16)
                if s == 0 and sub == SUB - 1:
                    pl.semaphore_signal(
                        credit, inc=1,
                        device_id=(credit_peer,),
                        device_id_type=pl.DeviceIdType.MESH,
                    )
                if not last:
                    if s == 1 and sub == 0:
                        pl.semaphore_wait(credit, 1)
                    start(ring, s + 1, sub)

            for sub in range(SUB):
                step("a", sub, comm_a, stage_a, pa[sub], 0, credit_a, left)
                step("b", sub, comm_b, stage_b, pb[sub], n2, credit_b, right)

    return pl.pallas_call(
        body,
        out_shape=jax.ShapeDtypeStruct((m_per, n), jnp.float32),
        in_specs=[
            pl.BlockSpec(memory_space=pltpu.VMEM),
            pl.BlockSpec(memory_space=pltpu.VMEM),
        ],
        out_specs=pl.BlockSpec(memory_space=pltpu.VMEM),
        scratch_shapes=[
            pltpu.VMEM((2, m_per, n2), jnp.bfloat16),
            pltpu.VMEM((2, m_per, n2), jnp.bfloat16),
            pltpu.VMEM((m_per, n2), jnp.bfloat16),
            pltpu.VMEM((m_per, n2), jnp.bfloat16),
            pltpu.SemaphoreType.DMA((N_HOP, SUB)),
            pltpu.SemaphoreType.DMA((N_HOP, SUB)),
            pltpu.SemaphoreType.DMA((N_HOP, SUB)),
            pltpu.SemaphoreType.DMA((N_HOP, SUB)),
            pltpu.SemaphoreType.REGULAR,
            pltpu.SemaphoreType.REGULAR,
        ],
        compiler_params=pltpu.CompilerParams(
            collective_id=0,
            vmem_limit_bytes=120 * 1024 * 1024,
        ),
    )(x, w_mat)
